# baseline (speedup 1.0000x reference)
import functools

import jax
import jax.numpy as jnp
from jax.experimental import pallas as pl
from jax.experimental.pallas import tpu as pltpu


def _paired_head_kernel(x_ref, w_ref, b_ref, o_ref):
    z = jnp.dot(x_ref[...], w_ref[...], preferred_element_type=jnp.float32)
    z = z + b_ref[0]
    p = pl.reciprocal(1.0 + jnp.exp(-z), approx=True)
    o_ref[...] = p.astype(o_ref.dtype)


@jax.jit
def _decision_head_fast(x, weight, bias):
    B, T, H = x.shape
    rows = B * T
    half = rows // 2

    x_p = x.reshape(half, 2 * H)
    w = weight.reshape(H, 1).astype(x.dtype)
    zeros = jnp.zeros_like(w)
    w_pair = jnp.concatenate(
        [jnp.concatenate([w, zeros], axis=0),
         jnp.concatenate([zeros, w], axis=0)], axis=1)
    b1 = bias.reshape((1,)).astype(jnp.float32)

    tr = min(2048, half)
    out = pl.pallas_call(
        _paired_head_kernel,
        out_shape=jax.ShapeDtypeStruct((half, 2), x.dtype),
        grid=(pl.cdiv(half, tr),),
        in_specs=[
            pl.BlockSpec((tr, 2 * H), lambda i: (i, 0)),
            pl.BlockSpec((2 * H, 2), lambda i: (0, 0)),
            pl.BlockSpec(memory_space=pltpu.MemorySpace.SMEM),
        ],
        out_specs=pl.BlockSpec((tr, 2), lambda i: (i, 0)),
        compiler_params=pltpu.CompilerParams(
            dimension_semantics=("parallel",),
        ),
    )(x_p, w_pair, b1)
    return out.reshape(B, T)


def kernel(x, weight, bias):
    return _decision_head_fast(x, weight, bias)


# speedup vs baseline: 1.9309x; 1.9309x over previous
import jax
import jax.numpy as jnp
from jax import lax
from jax.experimental import pallas as pl
from jax.experimental.pallas import tpu as pltpu


def _head_kernel(x_ref, w_ref, b_ref, o_ref):
    tb, T, H = x_ref.shape
    xf = x_ref[...].reshape(tb * T, H)
    z = lax.dot_general(w_ref[...], xf, (((1,), (1,)), ((), ())),
                        preferred_element_type=jnp.float32)
    z = z + b_ref[0]
    p = pl.reciprocal(1.0 + jnp.exp(-z), approx=True)
    o_ref[...] = p.reshape(tb, T).astype(o_ref.dtype)


@jax.jit
def _decision_head_fast(x, weight, bias):
    B, T, H = x.shape
    w = weight.reshape(1, H).astype(x.dtype)
    b1 = bias.reshape((1,)).astype(jnp.float32)

    tb = 8
    return pl.pallas_call(
        _head_kernel,
        out_shape=jax.ShapeDtypeStruct((B, T), x.dtype),
        grid=(pl.cdiv(B, tb),),
        in_specs=[
            pl.BlockSpec((tb, T, H), lambda i: (i, 0, 0)),
            pl.BlockSpec((1, H), lambda i: (0, 0)),
            pl.BlockSpec(memory_space=pltpu.MemorySpace.SMEM),
        ],
        out_specs=pl.BlockSpec((tb, T), lambda i: (i, 0)),
        compiler_params=pltpu.CompilerParams(
            dimension_semantics=("parallel",),
        ),
    )(x, w, b1)


def kernel(x, weight, bias):
    return _decision_head_fast(x, weight, bias)


# speedup vs baseline: 2.2695x; 1.1754x over previous
import jax
import jax.numpy as jnp
from jax import lax
from jax.experimental import pallas as pl
from jax.experimental.pallas import tpu as pltpu


def _head_kernel(x_ref, w_ref, b_ref, o_ref):
    tb, T, H = x_ref.shape
    xf = x_ref[...].reshape(tb * T, H)
    z = lax.dot_general(w_ref[...], xf, (((1,), (1,)), ((), ())),
                        preferred_element_type=jnp.float32)
    z = z + b_ref[0]
    p = pl.reciprocal(1.0 + jnp.exp(-z), approx=True)
    o_ref[...] = p.reshape(tb, T).astype(o_ref.dtype)


@jax.jit
def _decision_head_fast(x, weight, bias):
    B, T, H = x.shape
    w = weight.reshape(1, H).astype(x.dtype)
    b1 = bias.reshape((1,)).astype(jnp.float32)

    tb = 32
    return pl.pallas_call(
        _head_kernel,
        out_shape=jax.ShapeDtypeStruct((B, T), x.dtype),
        grid=(pl.cdiv(B, tb),),
        in_specs=[
            pl.BlockSpec((tb, T, H), lambda i: (i, 0, 0)),
            pl.BlockSpec((1, H), lambda i: (0, 0)),
            pl.BlockSpec(memory_space=pltpu.MemorySpace.SMEM),
        ],
        out_specs=pl.BlockSpec((tb, T), lambda i: (i, 0)),
        compiler_params=pltpu.CompilerParams(
            dimension_semantics=("arbitrary",),
        ),
    )(x, w, b1)


def kernel(x, weight, bias):
    return _decision_head_fast(x, weight, bias)
